# Initial kernel scaffold; baseline (speedup 1.0000x reference)
import jax
import jax.numpy as jnp
from jax.experimental import pallas as pl


def kernel(nodes_color, probas, edges_nn, clusters, h0, h1, h2, Wl0, Wr0, b0, Wl1, Wr1, b1, Wl2, Wr2, b2, mu):
    raise NotImplementedError("write your pallas kernel here")


# speedup vs baseline: 1.1764x; 1.1764x over previous
import jax
import jax.numpy as jnp
from jax import lax
from jax.experimental import pallas as pl

N = 10000
E = 160000
K = 30
ER = 1250


def _dotT(a, w):
    return lax.dot_general(a, w, (((1,), (1,)), ((), ())),
                           preferred_element_type=jnp.float32)


def _edge_body(ks, kd, a0, a1, a2, d0, d1, d2, w_o, s0_o):
    w_o[...] = (ks[...] == kd[...]).astype(jnp.float32)
    e0 = a0[...] - d0[...]
    e1 = a1[...] - d1[...]
    e2 = a2[...] - d2[...]
    s0_o[...] = jnp.exp((e0 * e0 + e1 * e1 + e2 * e2) * (-1.0 / 255.0))


_edge = pl.pallas_call(
    _edge_body,
    out_shape=(jax.ShapeDtypeStruct((ER, 128), jnp.float32),
               jax.ShapeDtypeStruct((ER, 128), jnp.float32)))


def _l0_body(agg, col, cnt, wl, wr, b, h, x1_o):
    mean = (agg[...] + col[...]) / cnt[...]
    z = jnp.maximum(_dotT(mean, wl[...]) + _dotT(col[...], wr[...]) + b[...],
                    0.0)
    x1_o[...] = (z + h[...]) * 0.5


_tc_l0 = pl.pallas_call(
    _l0_body, out_shape=jax.ShapeDtypeStruct((N, 64), jnp.float32))


_RB = 2000


def _l1_body(agg, x1, cnt, h, wl, wr, b, wl2, x2_o, y2_o):
    mean = (agg[...] + x1[...]) / cnt[...]
    z = jnp.maximum(_dotT(mean, wl[...]) + _dotT(x1[...], wr[...]) + b[...],
                    0.0)
    x2 = (z + h[...]) * 0.5
    x2_o[...] = x2
    y2_o[...] = _dotT(x2, wl2[...])


def _row_spec(d):
    return pl.BlockSpec((_RB, d), lambda i: (i, 0))


def _full_spec(shape):
    return pl.BlockSpec(shape, lambda i: tuple(0 for _ in shape))


_tc_l1 = pl.pallas_call(
    _l1_body,
    grid=(N // _RB,),
    in_specs=[_row_spec(64), _row_spec(64), _row_spec(1), _row_spec(512),
              _full_spec((512, 64)), _full_spec((512, 64)),
              _full_spec((1, 512)), _full_spec((256, 512))],
    out_specs=(_row_spec(512), _row_spec(256)),
    out_shape=(jax.ShapeDtypeStruct((N, 512), jnp.float32),
               jax.ShapeDtypeStruct((N, 256), jnp.float32)))


def _fin_body(agg, y2, x2, cnt, wr, b, h, mu, cg_o, x3_o):
    meanp = (agg[...] + y2[...]) / cnt[...]
    z = jnp.maximum(meanp + _dotT(x2[...], wr[...]) + b[...], 0.0)
    x3 = (z + h[...]) * 0.5
    x3_o[...] = x3
    m = mu[...]
    d2 = (jnp.sum(x3 * x3, axis=1, keepdims=True)
          + jnp.sum(m * m, axis=1)[None, :] - 2.0 * _dotT(x3, m))
    d2 = jnp.maximum(d2, 0.0)
    q = 1.0 / (1.0 + d2)
    q = q / jnp.sum(q, axis=1, keepdims=True)
    eq = jnp.exp(q - jnp.max(q, axis=1, keepdims=True))
    cg_o[...] = eq / jnp.sum(eq, axis=1, keepdims=True)


_tc_fin = pl.pallas_call(
    _fin_body,
    grid=(N // _RB,),
    in_specs=[_row_spec(256), _row_spec(256), _row_spec(512), _row_spec(1),
              _full_spec((256, 512)), _full_spec((1, 256)), _row_spec(256),
              _full_spec((K, 256))],
    out_specs=(_row_spec(K), _row_spec(256)),
    out_shape=(jax.ShapeDtypeStruct((N, K), jnp.float32),
               jax.ShapeDtypeStruct((N, 256), jnp.float32)))


def kernel(nodes_color, probas, edges_nn, clusters, h0, h1, h2,
           Wl0, Wr0, b0, Wl1, Wr1, b1, Wl2, Wr2, b2, mu):
    src = edges_nn[:, 0].astype(jnp.int32)
    dst = edges_nn[:, 1].astype(jnp.int32)
    key = (2 * jnp.argmax(clusters, axis=1).astype(jnp.int32)
           + (probas >= 0.5).astype(jnp.int32))
    cs = nodes_color[src]
    cd = nodes_color[dst]
    w2, s02 = _edge(key[src].reshape(ER, 128), key[dst].reshape(ER, 128),
                    cs[:, 0].reshape(ER, 128), cs[:, 1].reshape(ER, 128),
                    cs[:, 2].reshape(ER, 128), cd[:, 0].reshape(ER, 128),
                    cd[:, 1].reshape(ER, 128), cd[:, 2].reshape(ER, 128))
    w = w2.reshape(E)
    S0 = s02.reshape(E)

    cnt_col = (jax.ops.segment_sum(w, dst, num_segments=N) + 1.0).reshape(N, 1)
    agg0 = jax.ops.segment_sum(cs * w[:, None], dst, num_segments=N)
    x1 = _tc_l0(agg0, nodes_color, cnt_col, Wl0, Wr0, b0.reshape(1, 64), h0)

    agg1 = jax.ops.segment_sum(x1[src] * w[:, None], dst, num_segments=N)
    x2, y2 = _tc_l1(agg1, x1, cnt_col, h1, Wl1, Wr1, b1.reshape(1, 512), Wl2)

    agg2 = jax.ops.segment_sum(y2[src] * w[:, None], dst, num_segments=N)
    cg, x3 = _tc_fin(agg2, y2, x2, cnt_col, Wr2, b2.reshape(1, 256), h2, mu)
    return cg, x3, S0
